# Initial kernel scaffold; baseline (speedup 1.0000x reference)
import jax
import jax.numpy as jnp
from jax.experimental import pallas as pl


def kernel(val, mask):
    raise NotImplementedError("write your pallas kernel here")


# speedup vs baseline: 54.8798x; 54.8798x over previous
import functools

import jax
import jax.numpy as jnp
from jax import lax
from jax.experimental import pallas as pl

PH, PW = 2, 2


def _unpool_body(val_ref, mask_ref, out_ref, *, hb, W, C, H):
    Ho, Wo = H * PH, W * PW
    b = pl.program_id(0)
    h = pl.program_id(1)
    v = val_ref[0]
    m = mask_ref[0]
    y = lax.broadcasted_iota(jnp.int32, (hb, W, C), 0)
    x = lax.broadcasted_iota(jnp.int32, (hb, W, C), 1)
    c = lax.broadcasted_iota(jnp.int32, (hb, W, C), 2)
    yg = h * hb + y
    base = ((b * Ho + 2 * yg) * Wo + 2 * x) * C + c
    diff = m - base
    dy1 = diff >= Wo * C
    dx1 = (diff == C) | (diff == Wo * C + C)
    dy0 = jnp.logical_not(dy1)
    dx0 = jnp.logical_not(dx1)
    zero = jnp.zeros_like(v)
    for py, rowsel in ((0, dy0), (1, dy1)):
        a0 = jnp.where(rowsel & dx0, v, zero)
        a1 = jnp.where(rowsel & dx1, v, zero)
        out_ref[0, :, py] = jnp.concatenate([a0, a1], axis=-1)


def _unpool_tc(val, mask, interpret=False):
    B, H, W, C = val.shape
    Ho, Wo = H * PH, W * PW
    m = mask.astype(jnp.int32)
    hb = 16
    assert H % hb == 0
    body = functools.partial(_unpool_body, hb=hb, W=W, C=C, H=H)
    out = pl.pallas_call(
        body,
        grid=(B, H // hb),
        in_specs=[
            pl.BlockSpec((1, hb, W, C), lambda b, h: (b, h, 0, 0)),
            pl.BlockSpec((1, hb, W, C), lambda b, h: (b, h, 0, 0)),
        ],
        out_specs=pl.BlockSpec((1, hb, 2, W, 2 * C), lambda b, h: (b, h, 0, 0, 0)),
        out_shape=jax.ShapeDtypeStruct((B, H, 2, W, 2 * C), val.dtype),
        interpret=interpret,
    )(val, m)
    return out.reshape(B, Ho, Wo, C)


def kernel(val, mask):
    return _unpool_tc(val, mask)


# speedup vs baseline: 89.8879x; 1.6379x over previous
import functools

import jax
import jax.numpy as jnp
from jax import lax
from jax.experimental import pallas as pl

PH, PW = 2, 2


def _unpool_body(val_ref, mask_ref, out_ref, *, hb, W, C, H):
    Ho, Wo = H * PH, W * PW
    b = pl.program_id(0)
    h = pl.program_id(1)
    v = val_ref[0]
    m = mask_ref[0]
    vv = jnp.repeat(v, PH, axis=0)
    mm = jnp.repeat(m, PH, axis=0)
    Y = lax.broadcasted_iota(jnp.int32, (PH * hb, W, C), 0)
    X = lax.broadcasted_iota(jnp.int32, (PH * hb, W, C), 1)
    c = lax.broadcasted_iota(jnp.int32, (PH * hb, W, C), 2)
    Yg = h * (PH * hb) + Y
    zero = jnp.zeros_like(vv)
    for px in range(PW):
        oidx = ((b * Ho + Yg) * Wo + (PW * X + px)) * C + c
        out_ref[pl.ds(0, 1), :, pl.Slice(px, W, PW), :] = jnp.where(
            mm == oidx, vv, zero)[None]


def _unpool_tc(val, mask, interpret=False):
    B, H, W, C = val.shape
    Ho, Wo = H * PH, W * PW
    m = mask.astype(jnp.int32)
    hb = 16
    assert H % hb == 0
    body = functools.partial(_unpool_body, hb=hb, W=W, C=C, H=H)
    out = pl.pallas_call(
        body,
        grid=(B, H // hb),
        in_specs=[
            pl.BlockSpec((1, hb, W, C), lambda b, h: (b, h, 0, 0)),
            pl.BlockSpec((1, hb, W, C), lambda b, h: (b, h, 0, 0)),
        ],
        out_specs=pl.BlockSpec((1, PH * hb, Wo, C), lambda b, h: (b, h, 0, 0)),
        out_shape=jax.ShapeDtypeStruct((B, Ho, Wo, C), val.dtype),
        interpret=interpret,
    )(val, m)
    return out


def kernel(val, mask):
    return _unpool_tc(val, mask)


# speedup vs baseline: 90.8393x; 1.0106x over previous
import functools

import jax
import jax.numpy as jnp
from jax import lax
from jax.experimental import pallas as pl

PH, PW = 2, 2


def _unpool_body(val_ref, mask_ref, out_ref, *, hb, W, C, H):
    Ho, Wo = H * PH, W * PW
    b = pl.program_id(0)
    h = pl.program_id(1)
    v = val_ref[0]
    m = mask_ref[0]
    vv = jnp.repeat(v, PH, axis=0)
    mm = jnp.repeat(m, PH, axis=0)
    Y = lax.broadcasted_iota(jnp.int32, (PH * hb, W, C), 0)
    X = lax.broadcasted_iota(jnp.int32, (PH * hb, W, C), 1)
    c = lax.broadcasted_iota(jnp.int32, (PH * hb, W, C), 2)
    Yg = h * (PH * hb) + Y
    zero = jnp.zeros_like(vv)
    for px in range(PW):
        oidx = ((b * Ho + Yg) * Wo + (PW * X + px)) * C + c
        out_ref[pl.ds(0, 1), :, pl.Slice(px, W, PW), :] = jnp.where(
            mm == oidx, vv, zero)[None]


def _unpool_tc(val, mask, interpret=False, hb=16):
    B, H, W, C = val.shape
    Ho, Wo = H * PH, W * PW
    m = mask.astype(jnp.int32)
    assert H % hb == 0
    body = functools.partial(_unpool_body, hb=hb, W=W, C=C, H=H)
    out = pl.pallas_call(
        body,
        grid=(B, H // hb),
        in_specs=[
            pl.BlockSpec((1, hb, W, C), lambda b, h: (b, h, 0, 0)),
            pl.BlockSpec((1, hb, W, C), lambda b, h: (b, h, 0, 0)),
        ],
        out_specs=pl.BlockSpec((1, PH * hb, Wo, C), lambda b, h: (b, h, 0, 0)),
        out_shape=jax.ShapeDtypeStruct((B, Ho, Wo, C), val.dtype),
        interpret=interpret,
    )(val, m)
    return out


def kernel(val, mask):
    return _unpool_tc(val, mask, hb=28)


# speedup vs baseline: 322.1649x; 3.5465x over previous
import functools

import jax
import jax.numpy as jnp
from jax import lax
from jax.experimental import pallas as pl

PH, PW = 2, 2


def _unpool_body(val_ref, mask_ref, out_ref, *, hb, W, C, H):
    Ho, Wo = H * PH, W * PW
    b = pl.program_id(0)
    h = pl.program_id(1)
    v = val_ref[0]
    m = mask_ref[0]
    ix = lax.broadcasted_iota(jnp.int32, (PH * hb, C, Wo), 2) // PW
    vv = jnp.take_along_axis(jnp.repeat(v, PH, axis=0), ix, axis=2)
    mm = jnp.take_along_axis(jnp.repeat(m, PH, axis=0), ix, axis=2)
    Y = lax.broadcasted_iota(jnp.int32, (PH * hb, C, Wo), 0)
    c = lax.broadcasted_iota(jnp.int32, (PH * hb, C, Wo), 1)
    X = lax.broadcasted_iota(jnp.int32, (PH * hb, C, Wo), 2)
    Yg = h * (PH * hb) + Y
    oidx = ((b * Ho + Yg) * Wo + X) * C + c
    out_ref[0] = jnp.where(mm == oidx, vv, jnp.zeros_like(vv))


def _unpool_tc(val, mask, interpret=False, hb=16):
    B, H, W, C = val.shape
    Ho, Wo = H * PH, W * PW
    vt = val.transpose(0, 1, 3, 2)
    mt = mask.astype(jnp.int32).transpose(0, 1, 3, 2)
    assert H % hb == 0
    body = functools.partial(_unpool_body, hb=hb, W=W, C=C, H=H)
    out_t = pl.pallas_call(
        body,
        grid=(B, H // hb),
        in_specs=[
            pl.BlockSpec((1, hb, C, W), lambda b, h: (b, h, 0, 0)),
            pl.BlockSpec((1, hb, C, W), lambda b, h: (b, h, 0, 0)),
        ],
        out_specs=pl.BlockSpec((1, PH * hb, C, Wo), lambda b, h: (b, h, 0, 0)),
        out_shape=jax.ShapeDtypeStruct((B, Ho, C, Wo), val.dtype),
        interpret=interpret,
    )(vt, mt)
    return out_t.transpose(0, 1, 3, 2)


def kernel(val, mask):
    return _unpool_tc(val, mask, hb=16)


# speedup vs baseline: 365.5858x; 1.1348x over previous
import functools

import jax
import jax.numpy as jnp
from jax import lax
from jax.experimental import pallas as pl

PH, PW = 2, 2


def _unpool_body(val_ref, mask_ref, out_ref, *, hb, W, C, H):
    Ho, Wo = H * PH, W * PW
    b = pl.program_id(0)
    h = pl.program_id(1)
    v = val_ref[0]
    m = mask_ref[0]
    ix = lax.broadcasted_iota(jnp.int32, (PH * hb, C, Wo), 2) // PW
    vv = jnp.take_along_axis(jnp.repeat(v, PH, axis=0), ix, axis=2)
    mm = jnp.take_along_axis(jnp.repeat(m, PH, axis=0), ix, axis=2)
    Y = lax.broadcasted_iota(jnp.int32, (PH * hb, C, Wo), 0)
    c = lax.broadcasted_iota(jnp.int32, (PH * hb, C, Wo), 1)
    X = lax.broadcasted_iota(jnp.int32, (PH * hb, C, Wo), 2)
    Yg = h * (PH * hb) + Y
    oidx = ((b * Ho + Yg) * Wo + X) * C + c
    out_ref[0] = jnp.where(mm == oidx, vv, jnp.zeros_like(vv))


def _unpool_tc(val, mask, interpret=False, hb=16):
    B, H, W, C = val.shape
    Ho, Wo = H * PH, W * PW
    vt = val.transpose(0, 1, 3, 2)
    mt = mask.astype(jnp.int32).transpose(0, 1, 3, 2)
    assert H % hb == 0
    body = functools.partial(_unpool_body, hb=hb, W=W, C=C, H=H)
    out_t = pl.pallas_call(
        body,
        grid=(B, H // hb),
        in_specs=[
            pl.BlockSpec((1, hb, C, W), lambda b, h: (b, h, 0, 0)),
            pl.BlockSpec((1, hb, C, W), lambda b, h: (b, h, 0, 0)),
        ],
        out_specs=pl.BlockSpec((1, PH * hb, C, Wo), lambda b, h: (b, h, 0, 0)),
        out_shape=jax.ShapeDtypeStruct((B, Ho, C, Wo), val.dtype),
        interpret=interpret,
    )(vt, mt)
    return out_t.transpose(0, 1, 3, 2)


def kernel(val, mask):
    return _unpool_tc(val, mask, hb=28)


# speedup vs baseline: 385.0022x; 1.0531x over previous
import functools

import jax
import jax.numpy as jnp
from jax import lax
from jax.experimental import pallas as pl

PH, PW = 2, 2


def _unpool_body(val_ref, mask_ref, out_ref, *, hb, W, C, H):
    Ho, Wo = H * PH, W * PW
    b = pl.program_id(0)
    h = pl.program_id(1)
    v = val_ref[0]
    m = mask_ref[0]
    ix = lax.broadcasted_iota(jnp.int32, (PH * hb, C, Wo), 2) // PW
    vv = jnp.take_along_axis(jnp.repeat(v, PH, axis=0), ix, axis=2)
    mm = jnp.take_along_axis(jnp.repeat(m, PH, axis=0), ix, axis=2)
    Y = lax.broadcasted_iota(jnp.int32, (PH * hb, C, Wo), 0)
    c = lax.broadcasted_iota(jnp.int32, (PH * hb, C, Wo), 1)
    X = lax.broadcasted_iota(jnp.int32, (PH * hb, C, Wo), 2)
    Yg = h * (PH * hb) + Y
    oidx = ((b * Ho + Yg) * Wo + X) * C + c
    out_ref[0] = jnp.where(mm == oidx, vv, jnp.zeros_like(vv))


def _unpool_tc(val, mask, interpret=False, hb=16):
    B, H, W, C = val.shape
    Ho, Wo = H * PH, W * PW
    vt = val.transpose(0, 1, 3, 2)
    mt = mask.astype(jnp.int32).transpose(0, 1, 3, 2)
    assert H % hb == 0
    body = functools.partial(_unpool_body, hb=hb, W=W, C=C, H=H)
    out_t = pl.pallas_call(
        body,
        grid=(B, H // hb),
        in_specs=[
            pl.BlockSpec((1, hb, C, W), lambda b, h: (b, h, 0, 0)),
            pl.BlockSpec((1, hb, C, W), lambda b, h: (b, h, 0, 0)),
        ],
        out_specs=pl.BlockSpec((1, PH * hb, C, Wo), lambda b, h: (b, h, 0, 0)),
        out_shape=jax.ShapeDtypeStruct((B, Ho, C, Wo), val.dtype),
        interpret=interpret,
    )(vt, mt)
    return out_t.transpose(0, 1, 3, 2)


def kernel(val, mask):
    return _unpool_tc(val, mask, hb=56)
